# baseline (speedup 1.0000x reference)
import functools

import jax
import jax.numpy as jnp
from jax import lax
from jax.experimental import pallas as pl
from jax.experimental.pallas import tpu as pltpu
from jax.experimental.pallas import tpu_sc as plsc

B = 16384
OUT_D = 112
NC = 2
NS = 16
NW = NC * NS
RPW = B // NW
L = 16
RT = RPW // 128

_mesh = plsc.VectorSubcoreMesh(core_axis_name="c", subcore_axis_name="s")


@functools.partial(
    pl.kernel,
    out_type=jax.ShapeDtypeStruct((OUT_D // 8, B // 128, 8, 128), jnp.float32),
    mesh=_mesh,
    scratch_types=[
        pltpu.VMEM((RPW,), jnp.int32),
        pltpu.VMEM((RPW,), jnp.int32),
        pltpu.VMEM((RPW,), jnp.int32),
        pltpu.VMEM((RPW,), jnp.int32),
        pltpu.VMEM((RT, 4, 128), jnp.float32),
        pltpu.VMEM((RT, 4, 128), jnp.float32),
        pltpu.VMEM((RPW, 3), jnp.float32),
        pltpu.VMEM((RPW, 3), jnp.float32),
        pltpu.VMEM((RPW, 13), jnp.float32),
        pltpu.VMEM((RPW, 13), jnp.float32),
        pltpu.VMEM((400, 32), jnp.float32),
        pltpu.VMEM((8, 4), jnp.float32),
        pltpu.VMEM((OUT_D // 8, RT, 8, 128), jnp.float32),
        pltpu.SemaphoreType.DMA,
        pltpu.SemaphoreType.DMA,
    ],
    compiler_params=pltpu.CompilerParams(use_tc_tiling_on_sc=False,
                                         needs_layout_passes=False),
)
def _encode(p0c, p0b, p0k, p0a, p0j,
            p1c, p1b, p1k, p1a, p1j,
            at, jt, out,
            i0a, i0j, i1a, i1j,
            cs0, cs1, b0s, b1s, k0s, k1s, at_s, jt_s,
            blk, semi, semo):
    wid = lax.axis_index("s") * NC + lax.axis_index("c")
    base = wid * RPW
    sl = pl.ds(base, RPW)
    tsl = pl.ds(wid * RT, RT)

    cps = [
        pltpu.async_copy(p0a.at[sl], i0a, semi),
        pltpu.async_copy(p1a.at[sl], i1a, semi),
        pltpu.async_copy(p0j.at[sl], i0j, semi),
        pltpu.async_copy(p1j.at[sl], i1j, semi),
        pltpu.async_copy(p0c.at[tsl], cs0, semi),
        pltpu.async_copy(p1c.at[tsl], cs1, semi),
        pltpu.async_copy(p0b.at[sl], b0s, semi),
        pltpu.async_copy(p1b.at[sl], b1s, semi),
        pltpu.async_copy(p0k.at[sl], k0s, semi),
        pltpu.async_copy(p1k.at[sl], k1s, semi),
        pltpu.async_copy(at, at_s, semi),
        pltpu.async_copy(jt, jt_s, semi),
    ]
    for cp in cps:
        cp.wait()

    lanes = lax.iota(jnp.int32, L)

    @plsc.parallel_loop(0, RPW // L, unroll=2)
    def body(m):
        rv = lanes + m * L
        rt = m // 8
        ri = (m % 8) * L
        risl = pl.ds(ri, L)
        gsl = pl.ds(m * L, L)
        for half, csx, bsx, ksx, avx, jvx in (
            (0, cs0, b0s, k0s, i0a[gsl], i0j[gsl]),
            (7, cs1, b1s, k1s, i1a[gsl], i1j[gsl]),
        ):
            for c in range(4):
                o = half * 8 + c
                blk[o // 8, rt, o % 8, risl] = csx[rt, c, risl]
            for c in range(3):
                o = half * 8 + 4 + c
                blk[o // 8, rt, o % 8, risl] = plsc.load_gather(
                    bsx, [rv, jnp.full((L,), c, jnp.int32)])
            for c in range(13):
                o = half * 8 + 7 + c
                blk[o // 8, rt, o % 8, risl] = plsc.load_gather(
                    ksx, [rv, jnp.full((L,), c, jnp.int32)])
            for c in range(32):
                o = half * 8 + 20 + c
                blk[o // 8, rt, o % 8, risl] = plsc.load_gather(
                    at_s, [avx, jnp.full((L,), c, jnp.int32)])
            for c in range(4):
                o = half * 8 + 52 + c
                blk[o // 8, rt, o % 8, risl] = plsc.load_gather(
                    jt_s, [jvx, jnp.full((L,), c, jnp.int32)])

    o = pltpu.async_copy(blk, out.at[:, pl.ds(wid * RT, RT)], semo)
    o.wait()


def kernel(p0_continuous, p0_binary, p0_controller, p0_action, p0_jumps,
           p1_continuous, p1_binary, p1_controller, p1_action, p1_jumps,
           action_table, jumps_table):
    c0 = p0_continuous.reshape(128, 128, 4).transpose(0, 2, 1)
    c1 = p1_continuous.reshape(128, 128, 4).transpose(0, 2, 1)
    raw = _encode(c0, p0_binary, p0_controller,
                  p0_action.astype(jnp.int32), p0_jumps.astype(jnp.int32),
                  c1, p1_binary, p1_controller,
                  p1_action.astype(jnp.int32), p1_jumps.astype(jnp.int32),
                  action_table, jumps_table)
    return raw.transpose(1, 3, 0, 2).reshape(B, OUT_D)


# speedup vs baseline: 3.1752x; 3.1752x over previous
import functools

import jax
import jax.numpy as jnp
from jax import lax
from jax.experimental import pallas as pl
from jax.experimental.pallas import tpu as pltpu
from jax.experimental.pallas import tpu_sc as plsc

B = 16384
OUT_D = 112
NC = 2
NS = 16
NW = NC * NS
RPW = B // NW
L = 16
RT = RPW // 128

_mesh = plsc.VectorSubcoreMesh(core_axis_name="c", subcore_axis_name="s")


@functools.partial(
    pl.kernel,
    out_type=jax.ShapeDtypeStruct((OUT_D // 8, B // 128, 8, 128), jnp.float32),
    mesh=_mesh,
    scratch_types=[
        pltpu.VMEM((RPW,), jnp.int32),
        pltpu.VMEM((RPW,), jnp.int32),
        pltpu.VMEM((RPW,), jnp.int32),
        pltpu.VMEM((RPW,), jnp.int32),
        pltpu.VMEM((RT, 4, 128), jnp.float32),
        pltpu.VMEM((RT, 4, 128), jnp.float32),
        pltpu.VMEM((RT, 4, 128), jnp.float32),
        pltpu.VMEM((RT, 4, 128), jnp.float32),
        pltpu.VMEM((2, RT, 8, 128), jnp.float32),
        pltpu.VMEM((2, RT, 8, 128), jnp.float32),
        pltpu.VMEM((4, 4, 8, 128), jnp.float32),
        pltpu.VMEM((4, 128), jnp.float32),
        pltpu.VMEM((OUT_D // 8, RT, 8, 128), jnp.float32),
        pltpu.SemaphoreType.DMA,
        pltpu.SemaphoreType.DMA,
    ],
    compiler_params=pltpu.CompilerParams(use_tc_tiling_on_sc=False,
                                         needs_layout_passes=False),
)
def _encode(p0c, p0b, p0k, p0a, p0j,
            p1c, p1b, p1k, p1a, p1j,
            at, jt, out,
            i0a, i0j, i1a, i1j,
            cs0, cs1, bs0, bs1, ks0, ks1, at_s, jt_s,
            blk, semi, semo):
    wid = lax.axis_index("s") * NC + lax.axis_index("c")
    base = wid * RPW
    sl = pl.ds(base, RPW)
    tsl = pl.ds(wid * RT, RT)

    cps = [
        pltpu.async_copy(p0a.at[sl], i0a, semi),
        pltpu.async_copy(p1a.at[sl], i1a, semi),
        pltpu.async_copy(p0j.at[sl], i0j, semi),
        pltpu.async_copy(p1j.at[sl], i1j, semi),
        pltpu.async_copy(p0c.at[tsl], cs0, semi),
        pltpu.async_copy(p1c.at[tsl], cs1, semi),
        pltpu.async_copy(p0b.at[tsl], bs0, semi),
        pltpu.async_copy(p1b.at[tsl], bs1, semi),
        pltpu.async_copy(p0k.at[:, tsl], ks0, semi),
        pltpu.async_copy(p1k.at[:, tsl], ks1, semi),
        pltpu.async_copy(at, at_s, semi),
        pltpu.async_copy(jt, jt_s, semi),
    ]
    for cp in cps:
        cp.wait()

    @plsc.parallel_loop(0, RPW // L)
    def body(m):
        rt = m // 8
        ri = (m % 8) * L
        risl = pl.ds(ri, L)
        gsl = pl.ds(m * L, L)
        for half, csx, bsx, ksx, av, jv in (
            (0, cs0, bs0, ks0, i0a[gsl], i0j[gsl]),
            (7, cs1, bs1, ks1, i1a[gsl], i1j[gsl]),
        ):
            avr = jnp.right_shift(av, 7)
            avi = jnp.bitwise_and(av, 127)
            for c in range(4):
                o = half * 8 + c
                blk[o // 8, rt, o % 8, risl] = csx[rt, c, risl]
            for c in range(3):
                o = half * 8 + 4 + c
                blk[o // 8, rt, o % 8, risl] = bsx[rt, c, risl]
            for c in range(13):
                o = half * 8 + 7 + c
                blk[o // 8, rt, o % 8, risl] = ksx[c // 8, rt, c % 8, risl]
            for c in range(32):
                o = half * 8 + 20 + c
                blk[o // 8, rt, o % 8, risl] = plsc.load_gather(
                    at_s, [jnp.full((L,), c // 8, jnp.int32), avr,
                           jnp.full((L,), c % 8, jnp.int32), avi])
            for c in range(4):
                o = half * 8 + 52 + c
                blk[o // 8, rt, o % 8, risl] = plsc.load_gather(
                    jt_s, [jnp.full((L,), c, jnp.int32), jv])

    o = pltpu.async_copy(blk, out.at[:, pl.ds(wid * RT, RT)], semo)
    o.wait()


def _tiled_view_small(x, cols):
    if cols < 4:
        x = jnp.pad(x, ((0, 0), (0, 4 - cols)))
    return x.reshape(128, 128, 4).transpose(0, 2, 1)


def kernel(p0_continuous, p0_binary, p0_controller, p0_action, p0_jumps,
           p1_continuous, p1_binary, p1_controller, p1_action, p1_jumps,
           action_table, jumps_table):
    c0 = _tiled_view_small(p0_continuous, 4)
    c1 = _tiled_view_small(p1_continuous, 4)
    b0 = _tiled_view_small(p0_binary, 3)
    b1 = _tiled_view_small(p1_binary, 3)
    k0 = jnp.pad(p0_controller, ((0, 0), (0, 3))).reshape(128, 128, 2, 8).transpose(2, 0, 3, 1)
    k1 = jnp.pad(p1_controller, ((0, 0), (0, 3))).reshape(128, 128, 2, 8).transpose(2, 0, 3, 1)
    at4 = jnp.pad(action_table, ((0, 112), (0, 0))).reshape(4, 128, 4, 8).transpose(2, 0, 3, 1)
    jt2 = jnp.pad(jumps_table, ((0, 120), (0, 0))).T
    raw = _encode(c0, b0, k0,
                  p0_action.astype(jnp.int32), p0_jumps.astype(jnp.int32),
                  c1, b1, k1,
                  p1_action.astype(jnp.int32), p1_jumps.astype(jnp.int32),
                  at4, jt2)
    return raw.transpose(1, 3, 0, 2).reshape(B, OUT_D)
